# baseline (speedup 1.0000x reference)
import functools

import jax
import jax.numpy as jnp
from jax import lax
from jax.experimental import pallas as pl
from jax.experimental.pallas import tpu as pltpu
from jax.experimental.pallas import tpu_sc as plsc

_N = 10000
_M = 32
_AD = 64
_BD = 40
_FC = 128
_E = _N * _M

_BLK = 200
_GRID = _N // _BLK

_CH = 200


def _make_sc_gather():
    info = plsc.get_sparse_core_info()
    nw = info.num_cores * info.num_subcores
    b_per_w = _E // nw
    n_ch = b_per_w // _CH
    mesh = plsc.VectorSubcoreMesh(core_axis_name="c", subcore_axis_name="s")

    @functools.partial(
        pl.kernel,
        mesh=mesh,
        out_type=jax.ShapeDtypeStruct((_E, 2 * _AD), jnp.float32),
        scratch_types=[
            pltpu.VMEM((b_per_w,), jnp.int32),
            pltpu.VMEM((_CH, 2 * _AD), jnp.float32),
            pltpu.VMEM((_CH, 2 * _AD), jnp.float32),
            pltpu.SemaphoreType.DMA,
            pltpu.SemaphoreType.DMA,
        ],
    )
    def gather_kernel(table_hbm, idx_hbm, out_hbm, idx_v, buf_a, buf_b,
                      sem_a, sem_b):
        wid = lax.axis_index("s") * info.num_cores + lax.axis_index("c")
        base = wid * b_per_w
        pltpu.sync_copy(idx_hbm.at[pl.ds(base, b_per_w)], idx_v)

        def start(g, buf, sem):
            off = pl.multiple_of(g * _CH, 8)
            pltpu.async_copy(
                table_hbm.at[idx_v.at[pl.ds(off, _CH)]], buf, sem)

        def wait(buf, sem):
            pltpu.make_async_copy(
                table_hbm.at[idx_v.at[pl.ds(0, _CH)]], buf, sem).wait()

        def scatter(g, buf):
            off = pl.multiple_of(g * _CH, 8)
            pltpu.sync_copy(buf, out_hbm.at[pl.ds(base + off, _CH)])

        start(0, buf_a, sem_a)

        def body(gp, carry):
            g0 = gp * 2
            start(g0 + 1, buf_b, sem_b)
            wait(buf_a, sem_a)
            scatter(g0, buf_a)

            @pl.when(g0 + 2 < n_ch)
            def _():
                start(g0 + 2, buf_a, sem_a)

            wait(buf_b, sem_b)
            scatter(g0 + 1, buf_b)
            return carry

        lax.fori_loop(0, n_ch // 2, body, 0)

    return gather_kernel


_sc_gather_cache = []


def _sc_gather(table, idx):
    if not _sc_gather_cache:
        _sc_gather_cache.append(_make_sc_gather())
    return _sc_gather_cache[0](table, idx)


def _embed_body(af_ref, w_ref, b_ref, out_ref):
    out_ref[...] = (
        jnp.dot(af_ref[...], w_ref[...], preferred_element_type=jnp.float32, precision=lax.Precision.HIGHEST)
        + b_ref[...]
    )


def _embed(atom_feats, wemb, bemb):
    return pl.pallas_call(
        _embed_body,
        grid=(10,),
        in_specs=[
            pl.BlockSpec((_N // 10, 94), lambda i: (i, 0)),
            pl.BlockSpec((94, _AD), lambda i: (0, 0)),
            pl.BlockSpec((1, _AD), lambda i: (0, 0)),
        ],
        out_specs=pl.BlockSpec((_N // 10, _AD), lambda i: (i, 0)),
        out_shape=jax.ShapeDtypeStruct((_N, _AD), jnp.float32),
    )(atom_feats, wemb, bemb)


def _ymm_body(x_ref, wn_ref, out_ref):
    out_ref[...] = jnp.dot(x_ref[...], wn_ref[...],
                           preferred_element_type=jnp.float32, precision=lax.Precision.HIGHEST)


def _ymm(x, wn):
    return pl.pallas_call(
        _ymm_body,
        grid=(10,),
        in_specs=[
            pl.BlockSpec((_N // 10, _AD), lambda i: (i, 0)),
            pl.BlockSpec((_AD, 2 * _AD), lambda i: (0, 0)),
        ],
        out_specs=pl.BlockSpec((_N // 10, 2 * _AD), lambda i: (i, 0)),
        out_shape=jax.ShapeDtypeStruct((_N, 2 * _AD), jnp.float32),
    )(x, wn)


def _softplus(x):
    return jnp.maximum(x, 0.0) + jnp.log1p(jnp.exp(-jnp.abs(x)))


_LOG2E = 1.4426950408889634
_LN2 = 0.6931471805599453


def _conv_body(x_ref, gy_ref, d_ref, w_ref, b_ref, g1_ref, be1_ref,
               g2_ref, be2_ref, c_ref, out_ref):
    x = x_ref[...]
    w = w_ref[...]
    a = jnp.dot(x, w[0:_AD], preferred_element_type=jnp.float32, precision=lax.Precision.HIGHEST) + b_ref[...]
    d2 = d_ref[...]
    c = c_ref[...]
    cmat = jnp.concatenate([jnp.ones_like(c), -c], axis=0)
    t = jnp.dot(d2, cmat, preferred_element_type=jnp.float32, precision=lax.Precision.HIGHEST)
    bf = jnp.exp2(t * t * (-25.0 * _LOG2E))
    mm = gy_ref[...] + jnp.dot(bf, w[2 * _AD:],
                               preferred_element_type=jnp.float32, precision=lax.Precision.HIGHEST)
    p = (mm.reshape(_BLK, _M, 2 * _AD) + a[:, None, :]) \
        .reshape(_BLK * _M, 2 * _AD)
    ones_mat = jnp.full((2 * _AD, 2 * _AD), 1.0 / (2 * _AD),
                        dtype=jnp.float32)
    mu = jnp.dot(p, ones_mat, preferred_element_type=jnp.float32, precision=lax.Precision.HIGHEST)
    ctr = p - mu
    var = jnp.dot(ctr * ctr, ones_mat, preferred_element_type=jnp.float32, precision=lax.Precision.HIGHEST)
    nrm = ctr * lax.rsqrt(var + 1e-5) * g1_ref[...] + be1_ref[...]
    u = nrm[:, :_AD]
    v = nrm[:, _AD:]
    gate = 1.0 / (1.0 + jnp.exp2(u * -_LOG2E))
    core = _LN2 * jnp.log2(1.0 + jnp.exp2(v * _LOG2E))
    gc = jnp.mean((gate * core).reshape(_BLK, _M, _AD), axis=1)
    mu2 = jnp.mean(gc, axis=-1, keepdims=True)
    c2 = gc - mu2
    var2 = jnp.mean(c2 * c2, axis=-1, keepdims=True)
    out_ref[...] = x + c2 * lax.rsqrt(var2 + 1e-5) * g2_ref[...] \
        + be2_ref[...]


def _conv(x, gy, d_col, w, b, g1, be1, g2, be2, centers):
    return pl.pallas_call(
        _conv_body,
        grid=(_GRID,),
        in_specs=[
            pl.BlockSpec((_BLK, _AD), lambda i: (i, 0)),
            pl.BlockSpec((_BLK * _M, 2 * _AD), lambda i: (i, 0)),
            pl.BlockSpec((_BLK * _M, 2), lambda i: (i, 0)),
            pl.BlockSpec((2 * _AD + _BD, 2 * _AD), lambda i: (0, 0)),
            pl.BlockSpec((1, 2 * _AD), lambda i: (0, 0)),
            pl.BlockSpec((1, 2 * _AD), lambda i: (0, 0)),
            pl.BlockSpec((1, 2 * _AD), lambda i: (0, 0)),
            pl.BlockSpec((1, _AD), lambda i: (0, 0)),
            pl.BlockSpec((1, _AD), lambda i: (0, 0)),
            pl.BlockSpec((1, _BD), lambda i: (0, 0)),
        ],
        out_specs=pl.BlockSpec((_BLK, _AD), lambda i: (i, 0)),
        out_shape=jax.ShapeDtypeStruct((_N, _AD), jnp.float32),
    )(x, gy, d_col, w, b, g1, be1, g2, be2, centers)


def _head_body(x_ref, w1_ref, b1_ref, w2_ref, b2_ref, out_ref):
    pooled = jnp.mean(x_ref[...], axis=0, keepdims=True)
    h = jnp.dot(pooled, w1_ref[...], preferred_element_type=jnp.float32, precision=lax.Precision.HIGHEST) \
        + b1_ref[...]
    h = _softplus(h)
    out_ref[...] = jnp.dot(h, w2_ref[...], preferred_element_type=jnp.float32, precision=lax.Precision.HIGHEST) \
        + b2_ref[...]


def _head(x, w1, b1, w2, b2):
    return pl.pallas_call(
        _head_body,
        in_specs=[
            pl.BlockSpec((_N, _AD), lambda: (0, 0)),
            pl.BlockSpec((_AD, _FC), lambda: (0, 0)),
            pl.BlockSpec((1, _FC), lambda: (0, 0)),
            pl.BlockSpec((_FC, 1), lambda: (0, 0)),
            pl.BlockSpec((1, 1), lambda: (0, 0)),
        ],
        out_specs=pl.BlockSpec((1, 1), lambda: (0, 0)),
        out_shape=jax.ShapeDtypeStruct((1, 1), jnp.float32),
    )(x, w1, b1, w2, b2)


def kernel(atom_feats, bond_dists, nbr_indices, Wemb, bemb, centers,
           conv0_W, conv0_b, conv0_g1, conv0_beta1, conv0_g2, conv0_beta2,
           conv1_W, conv1_b, conv1_g1, conv1_beta1, conv1_g2, conv1_beta2,
           W1, b1, W2, b2):
    x = _embed(atom_feats, Wemb, bemb.reshape(1, _AD))
    nbr_flat = nbr_indices.reshape(_E)
    d_flat = bond_dists.reshape(_E, 1)
    d_col = jnp.concatenate([d_flat, jnp.ones_like(d_flat)], axis=1)
    c_row = centers.reshape(1, _BD)
    for w, b, g1, be1, g2, be2 in (
        (conv0_W, conv0_b, conv0_g1, conv0_beta1, conv0_g2, conv0_beta2),
        (conv1_W, conv1_b, conv1_g1, conv1_beta1, conv1_g2, conv1_beta2),
    ):
        y = _ymm(x, w[_AD:2 * _AD])
        gy = _sc_gather(y, nbr_flat)
        x = _conv(x, gy, d_col, w, b.reshape(1, -1), g1.reshape(1, -1),
                  be1.reshape(1, -1), g2.reshape(1, -1), be2.reshape(1, -1),
                  c_row)
    out = _head(x, W1, b1.reshape(1, _FC), W2, b2.reshape(1, 1))
    return out[0, 0]


# speedup vs baseline: 2.7756x; 2.7756x over previous
import functools

import jax
import jax.numpy as jnp
from jax import lax
from jax.experimental import pallas as pl
from jax.experimental.pallas import tpu as pltpu
from jax.experimental.pallas import tpu_sc as plsc

_N = 10000
_M = 32
_AD = 64
_BD = 40
_FC = 128
_E = _N * _M

_BLK = 200
_GRID = _N // _BLK

_CH = 200


def _make_sc_gather():
    info = plsc.get_sparse_core_info()
    nw = info.num_cores * info.num_subcores
    b_per_w = _E // nw
    n_ch = b_per_w // _CH
    mesh = plsc.VectorSubcoreMesh(core_axis_name="c", subcore_axis_name="s")

    @functools.partial(
        pl.kernel,
        mesh=mesh,
        out_type=jax.ShapeDtypeStruct((_E, 2 * _AD), jnp.float32),
        scratch_types=[
            pltpu.VMEM((b_per_w,), jnp.int32),
            pltpu.VMEM((_CH, 2 * _AD), jnp.float32),
            pltpu.VMEM((_CH, 2 * _AD), jnp.float32),
            pltpu.SemaphoreType.DMA,
            pltpu.SemaphoreType.DMA,
        ],
    )
    def gather_kernel(table_hbm, idx_hbm, out_hbm, idx_v, buf_a, buf_b,
                      sem_a, sem_b):
        wid = lax.axis_index("s") * info.num_cores + lax.axis_index("c")
        base = wid * b_per_w
        pltpu.sync_copy(idx_hbm.at[pl.ds(base, b_per_w)], idx_v)

        def start(g, buf, sem):
            off = pl.multiple_of(g * _CH, 8)
            pltpu.async_copy(
                table_hbm.at[idx_v.at[pl.ds(off, _CH)]], buf, sem)

        def wait(buf, sem):
            pltpu.make_async_copy(
                table_hbm.at[idx_v.at[pl.ds(0, _CH)]], buf, sem).wait()

        def scatter(g, buf):
            off = pl.multiple_of(g * _CH, 8)
            pltpu.sync_copy(buf, out_hbm.at[pl.ds(base + off, _CH)])

        start(0, buf_a, sem_a)

        def body(gp, carry):
            g0 = gp * 2
            start(g0 + 1, buf_b, sem_b)
            wait(buf_a, sem_a)
            scatter(g0, buf_a)

            @pl.when(g0 + 2 < n_ch)
            def _():
                start(g0 + 2, buf_a, sem_a)

            wait(buf_b, sem_b)
            scatter(g0 + 1, buf_b)
            return carry

        lax.fori_loop(0, n_ch // 2, body, 0)

    return gather_kernel


_sc_gather_cache = []


def _sc_gather(table, idx):
    if not _sc_gather_cache:
        _sc_gather_cache.append(_make_sc_gather())
    return _sc_gather_cache[0](table, idx)


def _embed_body(af_ref, w_ref, b_ref, out_ref):
    out_ref[...] = (
        jnp.dot(af_ref[...], w_ref[...], preferred_element_type=jnp.float32, precision=lax.Precision.HIGHEST)
        + b_ref[...]
    )


def _embed(atom_feats, wemb, bemb):
    return pl.pallas_call(
        _embed_body,
        grid=(10,),
        in_specs=[
            pl.BlockSpec((_N // 10, 94), lambda i: (i, 0)),
            pl.BlockSpec((94, _AD), lambda i: (0, 0)),
            pl.BlockSpec((1, _AD), lambda i: (0, 0)),
        ],
        out_specs=pl.BlockSpec((_N // 10, _AD), lambda i: (i, 0)),
        out_shape=jax.ShapeDtypeStruct((_N, _AD), jnp.float32),
    )(atom_feats, wemb, bemb)


def _ymm_body(x_ref, wn_ref, out_ref):
    out_ref[...] = jnp.dot(x_ref[...], wn_ref[...],
                           preferred_element_type=jnp.float32, precision=lax.Precision.HIGHEST)


def _ymm(x, wn):
    return pl.pallas_call(
        _ymm_body,
        grid=(10,),
        in_specs=[
            pl.BlockSpec((_N // 10, _AD), lambda i: (i, 0)),
            pl.BlockSpec((_AD, 2 * _AD), lambda i: (0, 0)),
        ],
        out_specs=pl.BlockSpec((_N // 10, 2 * _AD), lambda i: (i, 0)),
        out_shape=jax.ShapeDtypeStruct((_N, 2 * _AD), jnp.float32),
    )(x, wn)


def _softplus(x):
    return jnp.maximum(x, 0.0) + jnp.log1p(jnp.exp(-jnp.abs(x)))


_LOG2E = 1.4426950408889634
_LN2 = 0.6931471805599453


def _conv_body(x_ref, gy_ref, d_ref, w_ref, b_ref, g1_ref, be1_ref,
               g2_ref, be2_ref, c_ref, out_ref):
    x = x_ref[...]
    w = w_ref[...]
    a = jnp.dot(x, w[0:_AD], preferred_element_type=jnp.float32, precision=lax.Precision.HIGHEST) + b_ref[...]
    d = d_ref[...][:, :1]
    c = c_ref[...]
    t = d - c
    bf = jnp.exp2(t * t * (-25.0 * _LOG2E))
    mm = gy_ref[...] + jnp.dot(bf, w[2 * _AD:],
                               preferred_element_type=jnp.float32)
    p = (mm.reshape(_BLK, _M, 2 * _AD) + a[:, None, :]) \
        .reshape(_BLK * _M, 2 * _AD)
    ones_mat = jnp.full((2 * _AD, 2 * _AD), 1.0 / (2 * _AD),
                        dtype=jnp.float32)
    mu = jnp.dot(p, ones_mat, preferred_element_type=jnp.float32)
    ctr = p - mu
    var = jnp.dot(ctr * ctr, ones_mat, preferred_element_type=jnp.float32)
    nrm = ctr * lax.rsqrt(var + 1e-5) * g1_ref[...] + be1_ref[...]
    u = nrm[:, :_AD]
    v = nrm[:, _AD:]
    gate = 1.0 / (1.0 + jnp.exp2(u * -_LOG2E))
    core = _LN2 * jnp.log2(1.0 + jnp.exp2(v * _LOG2E))
    gc = jnp.mean((gate * core).reshape(_BLK, _M, _AD), axis=1)
    mu2 = jnp.mean(gc, axis=-1, keepdims=True)
    c2 = gc - mu2
    var2 = jnp.mean(c2 * c2, axis=-1, keepdims=True)
    out_ref[...] = x + c2 * lax.rsqrt(var2 + 1e-5) * g2_ref[...] \
        + be2_ref[...]


def _conv(x, gy, d_col, w, b, g1, be1, g2, be2, centers):
    return pl.pallas_call(
        _conv_body,
        grid=(_GRID,),
        in_specs=[
            pl.BlockSpec((_BLK, _AD), lambda i: (i, 0)),
            pl.BlockSpec((_BLK * _M, 2 * _AD), lambda i: (i, 0)),
            pl.BlockSpec((_BLK * _M, 2), lambda i: (i, 0)),
            pl.BlockSpec((2 * _AD + _BD, 2 * _AD), lambda i: (0, 0)),
            pl.BlockSpec((1, 2 * _AD), lambda i: (0, 0)),
            pl.BlockSpec((1, 2 * _AD), lambda i: (0, 0)),
            pl.BlockSpec((1, 2 * _AD), lambda i: (0, 0)),
            pl.BlockSpec((1, _AD), lambda i: (0, 0)),
            pl.BlockSpec((1, _AD), lambda i: (0, 0)),
            pl.BlockSpec((1, _BD), lambda i: (0, 0)),
        ],
        out_specs=pl.BlockSpec((_BLK, _AD), lambda i: (i, 0)),
        out_shape=jax.ShapeDtypeStruct((_N, _AD), jnp.float32),
    )(x, gy, d_col, w, b, g1, be1, g2, be2, centers)


def _head_body(x_ref, w1_ref, b1_ref, w2_ref, b2_ref, out_ref):
    pooled = jnp.mean(x_ref[...], axis=0, keepdims=True)
    h = jnp.dot(pooled, w1_ref[...], preferred_element_type=jnp.float32, precision=lax.Precision.HIGHEST) \
        + b1_ref[...]
    h = _softplus(h)
    out_ref[...] = jnp.dot(h, w2_ref[...], preferred_element_type=jnp.float32, precision=lax.Precision.HIGHEST) \
        + b2_ref[...]


def _head(x, w1, b1, w2, b2):
    return pl.pallas_call(
        _head_body,
        in_specs=[
            pl.BlockSpec((_N, _AD), lambda: (0, 0)),
            pl.BlockSpec((_AD, _FC), lambda: (0, 0)),
            pl.BlockSpec((1, _FC), lambda: (0, 0)),
            pl.BlockSpec((_FC, 1), lambda: (0, 0)),
            pl.BlockSpec((1, 1), lambda: (0, 0)),
        ],
        out_specs=pl.BlockSpec((1, 1), lambda: (0, 0)),
        out_shape=jax.ShapeDtypeStruct((1, 1), jnp.float32),
    )(x, w1, b1, w2, b2)


def kernel(atom_feats, bond_dists, nbr_indices, Wemb, bemb, centers,
           conv0_W, conv0_b, conv0_g1, conv0_beta1, conv0_g2, conv0_beta2,
           conv1_W, conv1_b, conv1_g1, conv1_beta1, conv1_g2, conv1_beta2,
           W1, b1, W2, b2):
    x = _embed(atom_feats, Wemb, bemb.reshape(1, _AD))
    nbr_flat = nbr_indices.reshape(_E)
    d_flat = bond_dists.reshape(_E, 1)
    d_col = jnp.concatenate([d_flat, jnp.ones_like(d_flat)], axis=1)
    c_row = centers.reshape(1, _BD)
    for w, b, g1, be1, g2, be2 in (
        (conv0_W, conv0_b, conv0_g1, conv0_beta1, conv0_g2, conv0_beta2),
        (conv1_W, conv1_b, conv1_g1, conv1_beta1, conv1_g2, conv1_beta2),
    ):
        y = _ymm(x, w[_AD:2 * _AD])
        gy = _sc_gather(y, nbr_flat)
        x = _conv(x, gy, d_col, w, b.reshape(1, -1), g1.reshape(1, -1),
                  be1.reshape(1, -1), g2.reshape(1, -1), be2.reshape(1, -1),
                  c_row)
    out = _head(x, W1, b1.reshape(1, _FC), W2, b2.reshape(1, 1))
    return out[0, 0]


# speedup vs baseline: 3.1390x; 1.1309x over previous
import functools

import jax
import jax.numpy as jnp
from jax import lax
from jax.experimental import pallas as pl
from jax.experimental.pallas import tpu as pltpu
from jax.experimental.pallas import tpu_sc as plsc

_N = 10000
_M = 32
_AD = 64
_BD = 40
_FC = 128
_E = _N * _M

_BLK = 200
_GRID = _N // _BLK

_CH = 200


def _make_sc_gather(n_edges):
    info = plsc.get_sparse_core_info()
    nw = info.num_cores * info.num_subcores
    b_per_w = n_edges // nw
    n_ch = b_per_w // _CH
    mesh = plsc.VectorSubcoreMesh(core_axis_name="c", subcore_axis_name="s")

    @functools.partial(
        pl.kernel,
        mesh=mesh,
        out_type=jax.ShapeDtypeStruct((n_edges, 2 * _AD), jnp.float32),
        scratch_types=[
            pltpu.VMEM((b_per_w,), jnp.int32),
            pltpu.VMEM((_CH, 2 * _AD), jnp.float32),
            pltpu.VMEM((_CH, 2 * _AD), jnp.float32),
            pltpu.SemaphoreType.DMA,
            pltpu.SemaphoreType.DMA,
        ],
    )
    def gather_kernel(table_hbm, idx_hbm, out_hbm, idx_v, buf_a, buf_b,
                      sem_a, sem_b):
        wid = lax.axis_index("s") * info.num_cores + lax.axis_index("c")
        base = wid * b_per_w
        pltpu.sync_copy(idx_hbm.at[pl.ds(base, b_per_w)], idx_v)

        def start(g, buf, sem):
            off = pl.multiple_of(g * _CH, 8)
            pltpu.async_copy(
                table_hbm.at[idx_v.at[pl.ds(off, _CH)]], buf, sem)

        def wait(buf, sem):
            pltpu.make_async_copy(
                table_hbm.at[idx_v.at[pl.ds(0, _CH)]], buf, sem).wait()

        def scatter(g, buf):
            off = pl.multiple_of(g * _CH, 8)
            pltpu.sync_copy(buf, out_hbm.at[pl.ds(base + off, _CH)])

        start(0, buf_a, sem_a)

        def body(gp, carry):
            g0 = gp * 2
            start(g0 + 1, buf_b, sem_b)
            wait(buf_a, sem_a)
            scatter(g0, buf_a)

            @pl.when(g0 + 2 < n_ch)
            def _():
                start(g0 + 2, buf_a, sem_a)

            wait(buf_b, sem_b)
            scatter(g0 + 1, buf_b)
            return carry

        lax.fori_loop(0, n_ch // 2, body, 0)
        if n_ch % 2 == 1:
            wait(buf_a, sem_a)
            scatter(n_ch - 1, buf_a)

    return gather_kernel


_sc_gather_cache = {}


def _sc_gather(table, idx):
    n_edges = idx.shape[0]
    if n_edges not in _sc_gather_cache:
        _sc_gather_cache[n_edges] = _make_sc_gather(n_edges)
    return _sc_gather_cache[n_edges](table, idx)


def _embed_body(af_ref, w_ref, b_ref, out_ref):
    out_ref[...] = (
        jnp.dot(af_ref[...], w_ref[...], preferred_element_type=jnp.float32, precision=lax.Precision.HIGHEST)
        + b_ref[...]
    )


def _embed(atom_feats, wemb, bemb):
    return pl.pallas_call(
        _embed_body,
        grid=(10,),
        in_specs=[
            pl.BlockSpec((_N // 10, 94), lambda i: (i, 0)),
            pl.BlockSpec((94, _AD), lambda i: (0, 0)),
            pl.BlockSpec((1, _AD), lambda i: (0, 0)),
        ],
        out_specs=pl.BlockSpec((_N // 10, _AD), lambda i: (i, 0)),
        out_shape=jax.ShapeDtypeStruct((_N, _AD), jnp.float32),
    )(atom_feats, wemb, bemb)


def _ymm_body(x_ref, wn_ref, out_ref):
    out_ref[...] = jnp.dot(x_ref[...], wn_ref[...],
                           preferred_element_type=jnp.float32, precision=lax.Precision.HIGHEST)


def _ymm(x, wn):
    return pl.pallas_call(
        _ymm_body,
        grid=(10,),
        in_specs=[
            pl.BlockSpec((_N // 10, _AD), lambda i: (i, 0)),
            pl.BlockSpec((_AD, 2 * _AD), lambda i: (0, 0)),
        ],
        out_specs=pl.BlockSpec((_N // 10, 2 * _AD), lambda i: (i, 0)),
        out_shape=jax.ShapeDtypeStruct((_N, 2 * _AD), jnp.float32),
    )(x, wn)


def _softplus(x):
    return jnp.maximum(x, 0.0) + jnp.log1p(jnp.exp(-jnp.abs(x)))


_LOG2E = 1.4426950408889634
_LN2 = 0.6931471805599453


def _conv_body(x_ref, gy_ref, d_ref, w_ref, b_ref, g1_ref, be1_ref,
               g2_ref, be2_ref, c_ref, out_ref):
    x = x_ref[...]
    w = w_ref[...]
    a = jnp.dot(x, w[0:_AD], preferred_element_type=jnp.float32, precision=lax.Precision.HIGHEST) + b_ref[...]
    d = d_ref[...]
    c = c_ref[...]
    t = d - c
    bf = jnp.exp2(t * t * (-25.0 * _LOG2E))
    mm = gy_ref[...] + jnp.dot(bf, w[2 * _AD:],
                               preferred_element_type=jnp.float32)
    p = (mm.reshape(_BLK, _M, 2 * _AD) + a[:, None, :]) \
        .reshape(_BLK * _M, 2 * _AD)
    ones_mat = jnp.full((2 * _AD, 2 * _AD), 1.0 / (2 * _AD),
                        dtype=jnp.float32)
    mu = jnp.dot(p, ones_mat, preferred_element_type=jnp.float32)
    ctr = p - mu
    var = jnp.dot(ctr * ctr, ones_mat, preferred_element_type=jnp.float32)
    nrm = ctr * lax.rsqrt(var + 1e-5) * g1_ref[...] + be1_ref[...]
    u = nrm[:, :_AD]
    v = nrm[:, _AD:]
    gate = 1.0 / (1.0 + jnp.exp2(u * -_LOG2E))
    core = _LN2 * jnp.log2(1.0 + jnp.exp2(v * _LOG2E))
    gc = jnp.mean((gate * core).reshape(_BLK, _M, _AD), axis=1)
    mu2 = jnp.mean(gc, axis=-1, keepdims=True)
    c2 = gc - mu2
    var2 = jnp.mean(c2 * c2, axis=-1, keepdims=True)
    out_ref[...] = x + c2 * lax.rsqrt(var2 + 1e-5) * g2_ref[...] \
        + be2_ref[...]


def _conv(x, gy, d_col, w, b, g1, be1, g2, be2, centers, off, nblk):
    return pl.pallas_call(
        _conv_body,
        grid=(nblk,),
        in_specs=[
            pl.BlockSpec((_BLK, _AD), lambda i, o=off: (i + o, 0)),
            pl.BlockSpec((_BLK * _M, 2 * _AD), lambda i: (i, 0)),
            pl.BlockSpec((_BLK * _M, 1), lambda i, o=off: (i + o, 0)),
            pl.BlockSpec((2 * _AD + _BD, 2 * _AD), lambda i: (0, 0)),
            pl.BlockSpec((1, 2 * _AD), lambda i: (0, 0)),
            pl.BlockSpec((1, 2 * _AD), lambda i: (0, 0)),
            pl.BlockSpec((1, 2 * _AD), lambda i: (0, 0)),
            pl.BlockSpec((1, _AD), lambda i: (0, 0)),
            pl.BlockSpec((1, _AD), lambda i: (0, 0)),
            pl.BlockSpec((1, _BD), lambda i: (0, 0)),
        ],
        out_specs=pl.BlockSpec((_BLK, _AD), lambda i: (i, 0)),
        out_shape=jax.ShapeDtypeStruct((nblk * _BLK, _AD), jnp.float32),
    )(x, gy, d_col, w, b, g1, be1, g2, be2, centers)


def _head_body(x_ref, w1_ref, b1_ref, w2_ref, b2_ref, out_ref):
    pooled = jnp.mean(x_ref[...], axis=0, keepdims=True)
    h = jnp.dot(pooled, w1_ref[...], preferred_element_type=jnp.float32, precision=lax.Precision.HIGHEST) \
        + b1_ref[...]
    h = _softplus(h)
    out_ref[...] = jnp.dot(h, w2_ref[...], preferred_element_type=jnp.float32, precision=lax.Precision.HIGHEST) \
        + b2_ref[...]


def _head(x, w1, b1, w2, b2):
    return pl.pallas_call(
        _head_body,
        in_specs=[
            pl.BlockSpec((_N, _AD), lambda: (0, 0)),
            pl.BlockSpec((_AD, _FC), lambda: (0, 0)),
            pl.BlockSpec((1, _FC), lambda: (0, 0)),
            pl.BlockSpec((_FC, 1), lambda: (0, 0)),
            pl.BlockSpec((1, 1), lambda: (0, 0)),
        ],
        out_specs=pl.BlockSpec((1, 1), lambda: (0, 0)),
        out_shape=jax.ShapeDtypeStruct((1, 1), jnp.float32),
    )(x, w1, b1, w2, b2)


def kernel(atom_feats, bond_dists, nbr_indices, Wemb, bemb, centers,
           conv0_W, conv0_b, conv0_g1, conv0_beta1, conv0_g2, conv0_beta2,
           conv1_W, conv1_b, conv1_g1, conv1_beta1, conv1_g2, conv1_beta2,
           W1, b1, W2, b2):
    x = _embed(atom_feats, Wemb, bemb.reshape(1, _AD))
    nbr_flat = nbr_indices.reshape(_E)
    d_col = bond_dists.reshape(_E, 1)
    c_row = centers.reshape(1, _BD)
    half_e = _E // 2
    half_blk = _GRID // 2
    for w, b, g1, be1, g2, be2 in (
        (conv0_W, conv0_b, conv0_g1, conv0_beta1, conv0_g2, conv0_beta2),
        (conv1_W, conv1_b, conv1_g1, conv1_beta1, conv1_g2, conv1_beta2),
    ):
        y = _ymm(x, w[_AD:2 * _AD])
        g0 = _sc_gather(y, nbr_flat[:half_e])
        g1h = _sc_gather(y, nbr_flat[half_e:])
        args = (w, b.reshape(1, -1), g1.reshape(1, -1), be1.reshape(1, -1),
                g2.reshape(1, -1), be2.reshape(1, -1), c_row)
        h0 = _conv(x, g0, d_col, *args, 0, half_blk)
        h1 = _conv(x, g1h, d_col, *args, half_blk, half_blk)
        x = jnp.concatenate([h0, h1], axis=0)
    out = _head(x, W1, b1.reshape(1, _FC), W2, b2.reshape(1, 1))
    return out[0, 0]


# speedup vs baseline: 3.9746x; 1.2662x over previous
import functools

import jax
import jax.numpy as jnp
from jax import lax
from jax.experimental import pallas as pl
from jax.experimental.pallas import tpu as pltpu
from jax.experimental.pallas import tpu_sc as plsc

_N = 10000
_M = 32
_AD = 64
_BD = 40
_FC = 128
_E = _N * _M

_BLK = 200
_GRID = _N // _BLK

_CH = 200


def _make_sc_gather(n_edges):
    info = plsc.get_sparse_core_info()
    nw = info.num_cores * info.num_subcores
    b_per_w = n_edges // nw
    n_ch = b_per_w // _CH
    mesh = plsc.VectorSubcoreMesh(core_axis_name="c", subcore_axis_name="s")

    @functools.partial(
        pl.kernel,
        mesh=mesh,
        out_type=jax.ShapeDtypeStruct((n_edges, 2 * _AD), jnp.float32),
        scratch_types=[
            pltpu.VMEM((b_per_w,), jnp.int32),
            pltpu.VMEM((_CH, 2 * _AD), jnp.float32),
            pltpu.VMEM((_CH, 2 * _AD), jnp.float32),
            pltpu.SemaphoreType.DMA,
            pltpu.SemaphoreType.DMA,
        ],
    )
    def gather_kernel(table_hbm, idx_hbm, out_hbm, idx_v, buf_a, buf_b,
                      sem_a, sem_b):
        wid = lax.axis_index("s") * info.num_cores + lax.axis_index("c")
        base = wid * b_per_w
        pltpu.sync_copy(idx_hbm.at[pl.ds(base, b_per_w)], idx_v)

        def start(g, buf, sem):
            off = pl.multiple_of(g * _CH, 8)
            pltpu.async_copy(
                table_hbm.at[idx_v.at[pl.ds(off, _CH)]], buf, sem)

        def wait(buf, sem):
            pltpu.make_async_copy(
                table_hbm.at[idx_v.at[pl.ds(0, _CH)]], buf, sem).wait()

        def scatter(g, buf):
            off = pl.multiple_of(g * _CH, 8)
            pltpu.sync_copy(buf, out_hbm.at[pl.ds(base + off, _CH)])

        start(0, buf_a, sem_a)

        def body(gp, carry):
            g0 = gp * 2
            start(g0 + 1, buf_b, sem_b)
            wait(buf_a, sem_a)
            scatter(g0, buf_a)

            @pl.when(g0 + 2 < n_ch)
            def _():
                start(g0 + 2, buf_a, sem_a)

            wait(buf_b, sem_b)
            scatter(g0 + 1, buf_b)
            return carry

        lax.fori_loop(0, n_ch // 2, body, 0)
        if n_ch % 2 == 1:
            wait(buf_a, sem_a)
            scatter(n_ch - 1, buf_a)

    return gather_kernel


_sc_gather_cache = {}


def _sc_gather(table, idx):
    n_edges = idx.shape[0]
    if n_edges not in _sc_gather_cache:
        _sc_gather_cache[n_edges] = _make_sc_gather(n_edges)
    return _sc_gather_cache[n_edges](table, idx)


def _embed_body(af_ref, w_ref, b_ref, out_ref):
    out_ref[...] = (
        jnp.dot(af_ref[...], w_ref[...], preferred_element_type=jnp.float32, precision=lax.Precision.HIGHEST)
        + b_ref[...]
    )


def _embed(atom_feats, wemb, bemb):
    return pl.pallas_call(
        _embed_body,
        grid=(10,),
        in_specs=[
            pl.BlockSpec((_N // 10, 94), lambda i: (i, 0)),
            pl.BlockSpec((94, _AD), lambda i: (0, 0)),
            pl.BlockSpec((1, _AD), lambda i: (0, 0)),
        ],
        out_specs=pl.BlockSpec((_N // 10, _AD), lambda i: (i, 0)),
        out_shape=jax.ShapeDtypeStruct((_N, _AD), jnp.float32),
    )(atom_feats, wemb, bemb)


def _ymm_body(x_ref, wn_ref, out_ref):
    out_ref[...] = jnp.dot(x_ref[...], wn_ref[...],
                           preferred_element_type=jnp.float32, precision=lax.Precision.HIGHEST)


def _ymm(x, wn):
    return pl.pallas_call(
        _ymm_body,
        grid=(10,),
        in_specs=[
            pl.BlockSpec((_N // 10, _AD), lambda i: (i, 0)),
            pl.BlockSpec((_AD, 2 * _AD), lambda i: (0, 0)),
        ],
        out_specs=pl.BlockSpec((_N // 10, 2 * _AD), lambda i: (i, 0)),
        out_shape=jax.ShapeDtypeStruct((_N, 2 * _AD), jnp.float32),
    )(x, wn)


def _softplus(x):
    return jnp.maximum(x, 0.0) + jnp.log1p(jnp.exp(-jnp.abs(x)))


_LOG2E = 1.4426950408889634
_LN2 = 0.6931471805599453


def _conv_body(x_ref, gy_ref, d_ref, w_ref, b_ref, g1_ref, be1_ref,
               g2_ref, be2_ref, c_ref, out_ref):
    x = x_ref[...]
    w = w_ref[...]
    a = jnp.dot(x, w[0:_AD], preferred_element_type=jnp.float32, precision=lax.Precision.HIGHEST) + b_ref[...]
    db = d_ref[...]
    a3 = jnp.broadcast_to(db[:, None, :], (_BLK, _M, _M))
    e3 = jnp.broadcast_to(
        jnp.eye(_M, dtype=jnp.float32)[None], (_BLK, _M, _M))
    d = jnp.sum((a3 * e3).reshape(_BLK * _M, _M), axis=-1, keepdims=True)
    c = c_ref[...]
    t = d - c
    bf = jnp.exp2(t * t * (-25.0 * _LOG2E))
    mm = gy_ref[...] + jnp.dot(bf, w[2 * _AD:],
                               preferred_element_type=jnp.float32)
    p = (mm.reshape(_BLK, _M, 2 * _AD) + a[:, None, :]) \
        .reshape(_BLK * _M, 2 * _AD)
    ones_mat = jnp.full((2 * _AD, 2 * _AD), 1.0 / (2 * _AD),
                        dtype=jnp.float32)
    mu = jnp.dot(p, ones_mat, preferred_element_type=jnp.float32)
    ctr = p - mu
    var = jnp.dot(ctr * ctr, ones_mat, preferred_element_type=jnp.float32)
    nrm = ctr * lax.rsqrt(var + 1e-5) * g1_ref[...] + be1_ref[...]
    u = nrm[:, :_AD]
    v = nrm[:, _AD:]
    gate = 1.0 / (1.0 + jnp.exp2(u * -_LOG2E))
    core = _LN2 * jnp.log2(1.0 + jnp.exp2(v * _LOG2E))
    gc = jnp.mean((gate * core).reshape(_BLK, _M, _AD), axis=1)
    mu2 = jnp.mean(gc, axis=-1, keepdims=True)
    c2 = gc - mu2
    var2 = jnp.mean(c2 * c2, axis=-1, keepdims=True)
    out_ref[...] = x + c2 * lax.rsqrt(var2 + 1e-5) * g2_ref[...] \
        + be2_ref[...]


def _conv(x, gy, d_col, w, b, g1, be1, g2, be2, centers, off, nblk):
    return pl.pallas_call(
        _conv_body,
        grid=(nblk,),
        in_specs=[
            pl.BlockSpec((_BLK, _AD), lambda i, o=off: (i + o, 0)),
            pl.BlockSpec((_BLK * _M, 2 * _AD), lambda i: (i, 0)),
            pl.BlockSpec((_BLK, _M), lambda i, o=off: (i + o, 0)),
            pl.BlockSpec((2 * _AD + _BD, 2 * _AD), lambda i: (0, 0)),
            pl.BlockSpec((1, 2 * _AD), lambda i: (0, 0)),
            pl.BlockSpec((1, 2 * _AD), lambda i: (0, 0)),
            pl.BlockSpec((1, 2 * _AD), lambda i: (0, 0)),
            pl.BlockSpec((1, _AD), lambda i: (0, 0)),
            pl.BlockSpec((1, _AD), lambda i: (0, 0)),
            pl.BlockSpec((1, _BD), lambda i: (0, 0)),
        ],
        out_specs=pl.BlockSpec((_BLK, _AD), lambda i: (i, 0)),
        out_shape=jax.ShapeDtypeStruct((nblk * _BLK, _AD), jnp.float32),
    )(x, gy, d_col, w, b, g1, be1, g2, be2, centers)


def _head_body(x_ref, w1_ref, b1_ref, w2_ref, b2_ref, out_ref):
    pooled = jnp.mean(x_ref[...], axis=0, keepdims=True)
    h = jnp.dot(pooled, w1_ref[...], preferred_element_type=jnp.float32, precision=lax.Precision.HIGHEST) \
        + b1_ref[...]
    h = _softplus(h)
    out_ref[...] = jnp.dot(h, w2_ref[...], preferred_element_type=jnp.float32, precision=lax.Precision.HIGHEST) \
        + b2_ref[...]


def _head(x, w1, b1, w2, b2):
    return pl.pallas_call(
        _head_body,
        in_specs=[
            pl.BlockSpec((_N, _AD), lambda: (0, 0)),
            pl.BlockSpec((_AD, _FC), lambda: (0, 0)),
            pl.BlockSpec((1, _FC), lambda: (0, 0)),
            pl.BlockSpec((_FC, 1), lambda: (0, 0)),
            pl.BlockSpec((1, 1), lambda: (0, 0)),
        ],
        out_specs=pl.BlockSpec((1, 1), lambda: (0, 0)),
        out_shape=jax.ShapeDtypeStruct((1, 1), jnp.float32),
    )(x, w1, b1, w2, b2)


def kernel(atom_feats, bond_dists, nbr_indices, Wemb, bemb, centers,
           conv0_W, conv0_b, conv0_g1, conv0_beta1, conv0_g2, conv0_beta2,
           conv1_W, conv1_b, conv1_g1, conv1_beta1, conv1_g2, conv1_beta2,
           W1, b1, W2, b2):
    x = _embed(atom_feats, Wemb, bemb.reshape(1, _AD))
    nbr_flat = nbr_indices.reshape(_E)
    d_col = bond_dists
    c_row = centers.reshape(1, _BD)
    half_e = _E // 2
    half_blk = _GRID // 2
    for w, b, g1, be1, g2, be2 in (
        (conv0_W, conv0_b, conv0_g1, conv0_beta1, conv0_g2, conv0_beta2),
        (conv1_W, conv1_b, conv1_g1, conv1_beta1, conv1_g2, conv1_beta2),
    ):
        y = _ymm(x, w[_AD:2 * _AD])
        g0 = _sc_gather(y, nbr_flat[:half_e])
        g1h = _sc_gather(y, nbr_flat[half_e:])
        args = (w, b.reshape(1, -1), g1.reshape(1, -1), be1.reshape(1, -1),
                g2.reshape(1, -1), be2.reshape(1, -1), c_row)
        h0 = _conv(x, g0, d_col, *args, 0, half_blk)
        h1 = _conv(x, g1h, d_col, *args, half_blk, half_blk)
        x = jnp.concatenate([h0, h1], axis=0)
    out = _head(x, W1, b1.reshape(1, _FC), W2, b2.reshape(1, 1))
    return out[0, 0]


# speedup vs baseline: 4.0860x; 1.0280x over previous
import functools

import jax
import jax.numpy as jnp
from jax import lax
from jax.experimental import pallas as pl
from jax.experimental.pallas import tpu as pltpu
from jax.experimental.pallas import tpu_sc as plsc

_N = 10000
_M = 32
_AD = 64
_BD = 40
_FC = 128
_E = _N * _M

_BLK = 200
_GRID = _N // _BLK

_CH = 200


def _make_sc_gather(n_edges):
    info = plsc.get_sparse_core_info()
    nw = info.num_cores * info.num_subcores
    b_per_w = n_edges // nw
    n_ch = b_per_w // _CH
    mesh = plsc.VectorSubcoreMesh(core_axis_name="c", subcore_axis_name="s")

    @functools.partial(
        pl.kernel,
        mesh=mesh,
        out_type=jax.ShapeDtypeStruct((n_edges, 2 * _AD), jnp.float32),
        scratch_types=[
            pltpu.VMEM((b_per_w,), jnp.int32),
            pltpu.VMEM((_CH, 2 * _AD), jnp.float32),
            pltpu.VMEM((_CH, 2 * _AD), jnp.float32),
            pltpu.SemaphoreType.DMA,
            pltpu.SemaphoreType.DMA,
        ],
    )
    def gather_kernel(table_hbm, idx_hbm, out_hbm, idx_v, buf_a, buf_b,
                      sem_a, sem_b):
        wid = lax.axis_index("s") * info.num_cores + lax.axis_index("c")
        base = wid * b_per_w
        pltpu.sync_copy(idx_hbm.at[pl.ds(base, b_per_w)], idx_v)

        def start(g, buf, sem):
            off = pl.multiple_of(g * _CH, 8)
            pltpu.async_copy(
                table_hbm.at[idx_v.at[pl.ds(off, _CH)]], buf, sem)

        def wait(buf, sem):
            pltpu.make_async_copy(
                table_hbm.at[idx_v.at[pl.ds(0, _CH)]], buf, sem).wait()

        def scatter(g, buf):
            off = pl.multiple_of(g * _CH, 8)
            pltpu.sync_copy(buf, out_hbm.at[pl.ds(base + off, _CH)])

        start(0, buf_a, sem_a)

        def body(gp, carry):
            g0 = gp * 2
            start(g0 + 1, buf_b, sem_b)
            wait(buf_a, sem_a)
            scatter(g0, buf_a)

            @pl.when(g0 + 2 < n_ch)
            def _():
                start(g0 + 2, buf_a, sem_a)

            wait(buf_b, sem_b)
            scatter(g0 + 1, buf_b)
            return carry

        lax.fori_loop(0, n_ch // 2, body, 0)
        if n_ch % 2 == 1:
            wait(buf_a, sem_a)
            scatter(n_ch - 1, buf_a)

    return gather_kernel


_sc_gather_cache = {}


def _sc_gather(table, idx):
    n_edges = idx.shape[0]
    if n_edges not in _sc_gather_cache:
        _sc_gather_cache[n_edges] = _make_sc_gather(n_edges)
    return _sc_gather_cache[n_edges](table, idx)


def _embed_body(af_ref, w_ref, b_ref, wn_ref, x_ref, y_ref):
    x = jnp.dot(af_ref[...], w_ref[...], preferred_element_type=jnp.float32,
                precision=lax.Precision.HIGHEST) + b_ref[...]
    x_ref[...] = x
    y_ref[...] = jnp.dot(x, wn_ref[...], preferred_element_type=jnp.float32,
                         precision=lax.Precision.HIGHEST)


def _embed(atom_feats, wemb, bemb, wn0):
    return pl.pallas_call(
        _embed_body,
        grid=(10,),
        in_specs=[
            pl.BlockSpec((_N // 10, 94), lambda i: (i, 0)),
            pl.BlockSpec((94, _AD), lambda i: (0, 0)),
            pl.BlockSpec((1, _AD), lambda i: (0, 0)),
            pl.BlockSpec((_AD, 2 * _AD), lambda i: (0, 0)),
        ],
        out_specs=[
            pl.BlockSpec((_N // 10, _AD), lambda i: (i, 0)),
            pl.BlockSpec((_N // 10, 2 * _AD), lambda i: (i, 0)),
        ],
        out_shape=[
            jax.ShapeDtypeStruct((_N, _AD), jnp.float32),
            jax.ShapeDtypeStruct((_N, 2 * _AD), jnp.float32),
        ],
    )(atom_feats, wemb, bemb, wn0)


def _ymm_body(x_ref, wn_ref, out_ref):
    out_ref[...] = jnp.dot(x_ref[...], wn_ref[...],
                           preferred_element_type=jnp.float32, precision=lax.Precision.HIGHEST)


def _ymm(x, wn):
    return pl.pallas_call(
        _ymm_body,
        grid=(10,),
        in_specs=[
            pl.BlockSpec((_N // 10, _AD), lambda i: (i, 0)),
            pl.BlockSpec((_AD, 2 * _AD), lambda i: (0, 0)),
        ],
        out_specs=pl.BlockSpec((_N // 10, 2 * _AD), lambda i: (i, 0)),
        out_shape=jax.ShapeDtypeStruct((_N, 2 * _AD), jnp.float32),
    )(x, wn)


def _softplus(x):
    return jnp.maximum(x, 0.0) + jnp.log1p(jnp.exp(-jnp.abs(x)))


_LOG2E = 1.4426950408889634
_LN2 = 0.6931471805599453


def _conv_body(x_ref, gy_ref, d_ref, w_ref, b_ref, g1_ref, be1_ref,
               g2_ref, be2_ref, c_ref, out_ref):
    x = x_ref[...]
    w = w_ref[...]
    a = jnp.dot(x, w[0:_AD], preferred_element_type=jnp.float32, precision=lax.Precision.HIGHEST) + b_ref[...]
    db = d_ref[...]
    a3 = jnp.broadcast_to(db[:, None, :], (_BLK, _M, _M))
    e3 = jnp.broadcast_to(
        jnp.eye(_M, dtype=jnp.float32)[None], (_BLK, _M, _M))
    d = jnp.sum((a3 * e3).reshape(_BLK * _M, _M), axis=-1, keepdims=True)
    c = c_ref[...]
    t = d - c
    bf = jnp.exp2(t * t * (-25.0 * _LOG2E))
    mm = gy_ref[...] + jnp.dot(bf, w[2 * _AD:],
                               preferred_element_type=jnp.float32)
    p = (mm.reshape(_BLK, _M, 2 * _AD) + a[:, None, :]) \
        .reshape(_BLK * _M, 2 * _AD)
    ones_mat = jnp.full((2 * _AD, 2 * _AD), 1.0 / (2 * _AD),
                        dtype=jnp.float32)
    mu = jnp.dot(p, ones_mat, preferred_element_type=jnp.float32)
    ctr = p - mu
    var = jnp.dot(ctr * ctr, ones_mat, preferred_element_type=jnp.float32)
    nrm = ctr * lax.rsqrt(var + 1e-5) * g1_ref[...] + be1_ref[...]
    u = nrm[:, :_AD]
    v = nrm[:, _AD:]
    gate = 1.0 / (1.0 + jnp.exp2(u * -_LOG2E))
    core = _LN2 * jnp.log2(1.0 + jnp.exp2(v * _LOG2E))
    gc = jnp.mean((gate * core).reshape(_BLK, _M, _AD), axis=1)
    mu2 = jnp.mean(gc, axis=-1, keepdims=True)
    c2 = gc - mu2
    var2 = jnp.mean(c2 * c2, axis=-1, keepdims=True)
    out_ref[...] = x + c2 * lax.rsqrt(var2 + 1e-5) * g2_ref[...] \
        + be2_ref[...]


def _conv(x, gy, d_col, w, b, g1, be1, g2, be2, centers, off, nblk):
    return pl.pallas_call(
        _conv_body,
        grid=(nblk,),
        in_specs=[
            pl.BlockSpec((_BLK, _AD), lambda i, o=off: (i + o, 0)),
            pl.BlockSpec((_BLK * _M, 2 * _AD), lambda i: (i, 0)),
            pl.BlockSpec((_BLK, _M), lambda i, o=off: (i + o, 0)),
            pl.BlockSpec((2 * _AD + _BD, 2 * _AD), lambda i: (0, 0)),
            pl.BlockSpec((1, 2 * _AD), lambda i: (0, 0)),
            pl.BlockSpec((1, 2 * _AD), lambda i: (0, 0)),
            pl.BlockSpec((1, 2 * _AD), lambda i: (0, 0)),
            pl.BlockSpec((1, _AD), lambda i: (0, 0)),
            pl.BlockSpec((1, _AD), lambda i: (0, 0)),
            pl.BlockSpec((1, _BD), lambda i: (0, 0)),
        ],
        out_specs=pl.BlockSpec((_BLK, _AD), lambda i: (i, 0)),
        out_shape=jax.ShapeDtypeStruct((nblk * _BLK, _AD), jnp.float32),
    )(x, gy, d_col, w, b, g1, be1, g2, be2, centers)


def _head_body(x_ref, w1_ref, b1_ref, w2_ref, b2_ref, out_ref):
    pooled = jnp.mean(x_ref[...], axis=0, keepdims=True)
    h = jnp.dot(pooled, w1_ref[...], preferred_element_type=jnp.float32, precision=lax.Precision.HIGHEST) \
        + b1_ref[...]
    h = _softplus(h)
    out_ref[...] = jnp.dot(h, w2_ref[...], preferred_element_type=jnp.float32, precision=lax.Precision.HIGHEST) \
        + b2_ref[...]


def _head(x, w1, b1, w2, b2):
    return pl.pallas_call(
        _head_body,
        in_specs=[
            pl.BlockSpec((_N, _AD), lambda: (0, 0)),
            pl.BlockSpec((_AD, _FC), lambda: (0, 0)),
            pl.BlockSpec((1, _FC), lambda: (0, 0)),
            pl.BlockSpec((_FC, 1), lambda: (0, 0)),
            pl.BlockSpec((1, 1), lambda: (0, 0)),
        ],
        out_specs=pl.BlockSpec((1, 1), lambda: (0, 0)),
        out_shape=jax.ShapeDtypeStruct((1, 1), jnp.float32),
    )(x, w1, b1, w2, b2)


def kernel(atom_feats, bond_dists, nbr_indices, Wemb, bemb, centers,
           conv0_W, conv0_b, conv0_g1, conv0_beta1, conv0_g2, conv0_beta2,
           conv1_W, conv1_b, conv1_g1, conv1_beta1, conv1_g2, conv1_beta2,
           W1, b1, W2, b2):
    x, y = _embed(atom_feats, Wemb, bemb.reshape(1, _AD),
                  conv0_W[_AD:2 * _AD])
    nbr_flat = nbr_indices.reshape(_E)
    d_col = bond_dists
    c_row = centers.reshape(1, _BD)
    parts = ((0, 2000), (2000, 4000), (6000, 4000))
    for li, (w, b, g1, be1, g2, be2) in enumerate((
        (conv0_W, conv0_b, conv0_g1, conv0_beta1, conv0_g2, conv0_beta2),
        (conv1_W, conv1_b, conv1_g1, conv1_beta1, conv1_g2, conv1_beta2),
    )):
        if li > 0:
            y = _ymm(x, w[_AD:2 * _AD])
        gys = [_sc_gather(y, nbr_flat[s * _M:(s + n) * _M])
               for s, n in parts]
        args = (w, b.reshape(1, -1), g1.reshape(1, -1), be1.reshape(1, -1),
                g2.reshape(1, -1), be2.reshape(1, -1), c_row)
        hs = [_conv(x, gy, d_col, *args, s // _BLK, n // _BLK)
              for gy, (s, n) in zip(gys, parts)]
        x = jnp.concatenate(hs, axis=0)
    out = _head(x, W1, b1.reshape(1, _FC), W2, b2.reshape(1, 1))
    return out[0, 0]


# speedup vs baseline: 4.1352x; 1.0121x over previous
import functools

import jax
import jax.numpy as jnp
from jax import lax
from jax.experimental import pallas as pl
from jax.experimental.pallas import tpu as pltpu
from jax.experimental.pallas import tpu_sc as plsc

_N = 10000
_M = 32
_AD = 64
_BD = 40
_FC = 128
_E = _N * _M

_BLK = 400
_GRID = _N // _BLK

_CH = 200


def _make_sc_gather(n_edges):
    info = plsc.get_sparse_core_info()
    nw = info.num_cores * info.num_subcores
    b_per_w = n_edges // nw
    n_ch = b_per_w // _CH
    mesh = plsc.VectorSubcoreMesh(core_axis_name="c", subcore_axis_name="s")

    @functools.partial(
        pl.kernel,
        mesh=mesh,
        out_type=jax.ShapeDtypeStruct((n_edges, 2 * _AD), jnp.float32),
        scratch_types=[
            pltpu.VMEM((b_per_w,), jnp.int32),
            pltpu.VMEM((_CH, 2 * _AD), jnp.float32),
            pltpu.VMEM((_CH, 2 * _AD), jnp.float32),
            pltpu.SemaphoreType.DMA,
            pltpu.SemaphoreType.DMA,
        ],
    )
    def gather_kernel(table_hbm, idx_hbm, out_hbm, idx_v, buf_a, buf_b,
                      sem_a, sem_b):
        wid = lax.axis_index("s") * info.num_cores + lax.axis_index("c")
        base = wid * b_per_w
        pltpu.sync_copy(idx_hbm.at[pl.ds(base, b_per_w)], idx_v)

        def start(g, buf, sem):
            off = pl.multiple_of(g * _CH, 8)
            pltpu.async_copy(
                table_hbm.at[idx_v.at[pl.ds(off, _CH)]], buf, sem)

        def wait(buf, sem):
            pltpu.make_async_copy(
                table_hbm.at[idx_v.at[pl.ds(0, _CH)]], buf, sem).wait()

        def scatter(g, buf):
            off = pl.multiple_of(g * _CH, 8)
            pltpu.sync_copy(buf, out_hbm.at[pl.ds(base + off, _CH)])

        start(0, buf_a, sem_a)

        def body(gp, carry):
            g0 = gp * 2
            start(g0 + 1, buf_b, sem_b)
            wait(buf_a, sem_a)
            scatter(g0, buf_a)

            @pl.when(g0 + 2 < n_ch)
            def _():
                start(g0 + 2, buf_a, sem_a)

            wait(buf_b, sem_b)
            scatter(g0 + 1, buf_b)
            return carry

        lax.fori_loop(0, n_ch // 2, body, 0)
        if n_ch % 2 == 1:
            wait(buf_a, sem_a)
            scatter(n_ch - 1, buf_a)

    return gather_kernel


_sc_gather_cache = {}


def _sc_gather(table, idx):
    n_edges = idx.shape[0]
    if n_edges not in _sc_gather_cache:
        _sc_gather_cache[n_edges] = _make_sc_gather(n_edges)
    return _sc_gather_cache[n_edges](table, idx)


def _embed_body(af_ref, w_ref, b_ref, wn_ref, x_ref, y_ref):
    x = jnp.dot(af_ref[...], w_ref[...], preferred_element_type=jnp.float32,
                precision=lax.Precision.HIGHEST) + b_ref[...]
    x_ref[...] = x
    y_ref[...] = jnp.dot(x, wn_ref[...], preferred_element_type=jnp.float32,
                         precision=lax.Precision.HIGHEST)


def _embed(atom_feats, wemb, bemb, wn0):
    return pl.pallas_call(
        _embed_body,
        grid=(10,),
        in_specs=[
            pl.BlockSpec((_N // 10, 94), lambda i: (i, 0)),
            pl.BlockSpec((94, _AD), lambda i: (0, 0)),
            pl.BlockSpec((1, _AD), lambda i: (0, 0)),
            pl.BlockSpec((_AD, 2 * _AD), lambda i: (0, 0)),
        ],
        out_specs=[
            pl.BlockSpec((_N // 10, _AD), lambda i: (i, 0)),
            pl.BlockSpec((_N // 10, 2 * _AD), lambda i: (i, 0)),
        ],
        out_shape=[
            jax.ShapeDtypeStruct((_N, _AD), jnp.float32),
            jax.ShapeDtypeStruct((_N, 2 * _AD), jnp.float32),
        ],
    )(atom_feats, wemb, bemb, wn0)


def _ymm_body(x_ref, wn_ref, out_ref):
    out_ref[...] = jnp.dot(x_ref[...], wn_ref[...],
                           preferred_element_type=jnp.float32, precision=lax.Precision.HIGHEST)


def _ymm(x, wn):
    return pl.pallas_call(
        _ymm_body,
        grid=(10,),
        in_specs=[
            pl.BlockSpec((_N // 10, _AD), lambda i: (i, 0)),
            pl.BlockSpec((_AD, 2 * _AD), lambda i: (0, 0)),
        ],
        out_specs=pl.BlockSpec((_N // 10, 2 * _AD), lambda i: (i, 0)),
        out_shape=jax.ShapeDtypeStruct((_N, 2 * _AD), jnp.float32),
    )(x, wn)


def _softplus(x):
    return jnp.maximum(x, 0.0) + jnp.log1p(jnp.exp(-jnp.abs(x)))


_LOG2E = 1.4426950408889634
_LN2 = 0.6931471805599453


def _conv_body(x_ref, gy_ref, d_ref, w_ref, b_ref, g1_ref, be1_ref,
               g2_ref, be2_ref, c_ref, out_ref):
    x = x_ref[...]
    w = w_ref[...]
    a = jnp.dot(x, w[0:_AD], preferred_element_type=jnp.float32, precision=lax.Precision.HIGHEST) + b_ref[...]
    db = d_ref[...]
    a3 = jnp.broadcast_to(db[:, None, :], (_BLK, _M, _M))
    e3 = jnp.broadcast_to(
        jnp.eye(_M, dtype=jnp.float32)[None], (_BLK, _M, _M))
    d = jnp.sum((a3 * e3).reshape(_BLK * _M, _M), axis=-1, keepdims=True)
    c = c_ref[...]
    t = d - c
    bf = jnp.exp2(t * t * (-25.0 * _LOG2E))
    mm = gy_ref[...] + jnp.dot(bf, w[2 * _AD:],
                               preferred_element_type=jnp.float32)
    p = (mm.reshape(_BLK, _M, 2 * _AD) + a[:, None, :]) \
        .reshape(_BLK * _M, 2 * _AD)
    ones_mat = jnp.full((2 * _AD, 2 * _AD), 1.0 / (2 * _AD),
                        dtype=jnp.float32)
    mu = jnp.dot(p, ones_mat, preferred_element_type=jnp.float32)
    ctr = p - mu
    var = jnp.dot(ctr * ctr, ones_mat, preferred_element_type=jnp.float32)
    nrm = ctr * lax.rsqrt(var + 1e-5) * g1_ref[...] + be1_ref[...]
    u = nrm[:, :_AD]
    v = nrm[:, _AD:]
    gate = 1.0 / (1.0 + jnp.exp2(u * -_LOG2E))
    core = _LN2 * jnp.log2(1.0 + jnp.exp2(v * _LOG2E))
    gc = jnp.mean((gate * core).reshape(_BLK, _M, _AD), axis=1)
    mu2 = jnp.mean(gc, axis=-1, keepdims=True)
    c2 = gc - mu2
    var2 = jnp.mean(c2 * c2, axis=-1, keepdims=True)
    out_ref[...] = x + c2 * lax.rsqrt(var2 + 1e-5) * g2_ref[...] \
        + be2_ref[...]


def _conv(x, gy, d_col, w, b, g1, be1, g2, be2, centers, off, nblk):
    return pl.pallas_call(
        _conv_body,
        grid=(nblk,),
        in_specs=[
            pl.BlockSpec((_BLK, _AD), lambda i, o=off: (i + o, 0)),
            pl.BlockSpec((_BLK * _M, 2 * _AD), lambda i: (i, 0)),
            pl.BlockSpec((_BLK, _M), lambda i, o=off: (i + o, 0)),
            pl.BlockSpec((2 * _AD + _BD, 2 * _AD), lambda i: (0, 0)),
            pl.BlockSpec((1, 2 * _AD), lambda i: (0, 0)),
            pl.BlockSpec((1, 2 * _AD), lambda i: (0, 0)),
            pl.BlockSpec((1, 2 * _AD), lambda i: (0, 0)),
            pl.BlockSpec((1, _AD), lambda i: (0, 0)),
            pl.BlockSpec((1, _AD), lambda i: (0, 0)),
            pl.BlockSpec((1, _BD), lambda i: (0, 0)),
        ],
        out_specs=pl.BlockSpec((_BLK, _AD), lambda i: (i, 0)),
        out_shape=jax.ShapeDtypeStruct((nblk * _BLK, _AD), jnp.float32),
    )(x, gy, d_col, w, b, g1, be1, g2, be2, centers)


def _head_body(x_ref, w1_ref, b1_ref, w2_ref, b2_ref, out_ref):
    pooled = jnp.mean(x_ref[...], axis=0, keepdims=True)
    h = jnp.dot(pooled, w1_ref[...], preferred_element_type=jnp.float32, precision=lax.Precision.HIGHEST) \
        + b1_ref[...]
    h = _softplus(h)
    out_ref[...] = jnp.dot(h, w2_ref[...], preferred_element_type=jnp.float32, precision=lax.Precision.HIGHEST) \
        + b2_ref[...]


def _head(x, w1, b1, w2, b2):
    return pl.pallas_call(
        _head_body,
        in_specs=[
            pl.BlockSpec((_N, _AD), lambda: (0, 0)),
            pl.BlockSpec((_AD, _FC), lambda: (0, 0)),
            pl.BlockSpec((1, _FC), lambda: (0, 0)),
            pl.BlockSpec((_FC, 1), lambda: (0, 0)),
            pl.BlockSpec((1, 1), lambda: (0, 0)),
        ],
        out_specs=pl.BlockSpec((1, 1), lambda: (0, 0)),
        out_shape=jax.ShapeDtypeStruct((1, 1), jnp.float32),
    )(x, w1, b1, w2, b2)


def kernel(atom_feats, bond_dists, nbr_indices, Wemb, bemb, centers,
           conv0_W, conv0_b, conv0_g1, conv0_beta1, conv0_g2, conv0_beta2,
           conv1_W, conv1_b, conv1_g1, conv1_beta1, conv1_g2, conv1_beta2,
           W1, b1, W2, b2):
    x, y = _embed(atom_feats, Wemb, bemb.reshape(1, _AD),
                  conv0_W[_AD:2 * _AD])
    nbr_flat = nbr_indices.reshape(_E)
    d_col = bond_dists
    c_row = centers.reshape(1, _BD)
    parts = ((0, 2000), (2000, 4000), (6000, 4000))
    for li, (w, b, g1, be1, g2, be2) in enumerate((
        (conv0_W, conv0_b, conv0_g1, conv0_beta1, conv0_g2, conv0_beta2),
        (conv1_W, conv1_b, conv1_g1, conv1_beta1, conv1_g2, conv1_beta2),
    )):
        if li > 0:
            y = _ymm(x, w[_AD:2 * _AD])
        gys = [_sc_gather(y, nbr_flat[s * _M:(s + n) * _M])
               for s, n in parts]
        args = (w, b.reshape(1, -1), g1.reshape(1, -1), be1.reshape(1, -1),
                g2.reshape(1, -1), be2.reshape(1, -1), c_row)
        hs = [_conv(x, gy, d_col, *args, s // _BLK, n // _BLK)
              for gy, (s, n) in zip(gys, parts)]
        x = jnp.concatenate(hs, axis=0)
    out = _head(x, W1, b1.reshape(1, _FC), W2, b2.reshape(1, 1))
    return out[0, 0]
